# Initial kernel scaffold; baseline (speedup 1.0000x reference)
import jax
import jax.numpy as jnp
from jax.experimental import pallas as pl


def kernel(feat, edge_index, diff_edge_index, edge_weight, W1, b1, a_conv1, gamma1, beta1, a_act1, W2, b2, a_conv2, gamma2, beta2, a_act2):
    raise NotImplementedError("write your pallas kernel here")


# speedup vs baseline: 1.9887x; 1.9887x over previous
import functools

import jax
import jax.numpy as jnp
from jax import lax
from jax.experimental import pallas as pl
from jax.experimental.pallas import tpu as pltpu
from jax.experimental.pallas import tpu_sc as plsc

N = 10000
D = 128
E = 320000
NC = 2
NS = 16
NW = NC * NS
N_PAD = 10240
E_PAD = 327680
EPW = E_PAD // NW
K = 128
RPT = N_PAD // NS
BN_EPS = 1e-5

_mesh = plsc.VectorSubcoreMesh(
    core_axis_name="c", subcore_axis_name="s", num_cores=NC, num_subcores=NS)


def _hist_body(idx_hbm, ones_hbm, zeros_hbm, out_hbm, idx_v, ones_v, acc, sem):
    cid = lax.axis_index("c")
    sid = lax.axis_index("s")
    r0 = sid * RPT
    pltpu.sync_copy(zeros_hbm.at[pl.ds(r0, RPT)], acc.at[pl.ds(r0, RPT)])
    pltpu.sync_copy(ones_hbm, ones_v)
    plsc.subcore_barrier()
    ebase = (cid * NS + sid) * EPW

    def chunk(i, carry):
        base = ebase + i * K
        pltpu.sync_copy(idx_hbm.at[pl.ds(base, K)], idx_v)
        pltpu.sync_copy(ones_v, acc.at[idx_v], add=True)
        return carry

    lax.fori_loop(0, EPW // K, chunk, 0)
    plsc.subcore_barrier()
    pltpu.sync_copy(acc.at[pl.ds(r0, RPT)],
                    out_hbm.at[pl.ds(cid * N_PAD + r0, RPT)])


_hist_kernel = functools.partial(
    pl.kernel,
    out_type=jax.ShapeDtypeStruct((NC * N_PAD, D), jnp.float32),
    mesh=_mesh,
    scratch_types=[
        pltpu.VMEM((K,), jnp.int32),
        pltpu.VMEM((K, D), jnp.float32),
        pltpu.VMEM_SHARED((N_PAD, D), jnp.float32),
        pltpu.SemaphoreType.DMA,
    ])(_hist_body)


def _make_agg(weighted):
    def body(table_hbm, src_hbm, dst_hbm, ew_hbm, zeros_hbm, out_hbm,
             src_v, dst_v, rows_v, w_v, acc, sem):
        cid = lax.axis_index("c")
        sid = lax.axis_index("s")
        r0 = sid * RPT
        pltpu.sync_copy(zeros_hbm.at[pl.ds(r0, RPT)], acc.at[pl.ds(r0, RPT)])
        plsc.subcore_barrier()
        ebase = (cid * NS + sid) * EPW

        def chunk(i, carry):
            base = ebase + i * K
            pltpu.sync_copy(src_hbm.at[pl.ds(base, K)], src_v)
            pltpu.sync_copy(dst_hbm.at[pl.ds(base, K)], dst_v)
            pltpu.async_copy(table_hbm.at[src_v], rows_v, sem).wait()
            if weighted:
                pltpu.sync_copy(ew_hbm.at[pl.ds(base, K)], w_v)

                def wmul(g, c):
                    w16 = w_v[pl.ds(g * 16, 16)]
                    for l in range(16):
                        j = g * 16 + l
                        wl = w16[l]
                        for t in range(D // 16):
                            rows_v[j, pl.ds(t * 16, 16)] = (
                                rows_v[j, pl.ds(t * 16, 16)] * wl)
                    return c

                lax.fori_loop(0, K // 16, wmul, 0)
            pltpu.sync_copy(rows_v, acc.at[dst_v], add=True)
            return carry

        lax.fori_loop(0, EPW // K, chunk, 0)
        plsc.subcore_barrier()
        pltpu.sync_copy(acc.at[pl.ds(r0, RPT)],
                        out_hbm.at[pl.ds(cid * N_PAD + r0, RPT)])

    return functools.partial(
        pl.kernel,
        out_type=jax.ShapeDtypeStruct((NC * N_PAD, D), jnp.float32),
        mesh=_mesh,
        scratch_types=[
            pltpu.VMEM((K,), jnp.int32),
            pltpu.VMEM((K,), jnp.int32),
            pltpu.VMEM((K, D), jnp.float32),
            pltpu.VMEM((K,), jnp.float32),
            pltpu.VMEM_SHARED((N_PAD, D), jnp.float32),
            pltpu.SemaphoreType.DMA,
        ])(body)


_agg_plain = _make_agg(False)
_agg_weighted = _make_agg(True)


def _prep_body(feat_ref, dego_ref, keep0_ref, keep1_ref,
               t1_ref, t2_ref, t3_ref):
    x = feat_ref[...]
    deg_out = dego_ref[0, :, 0:1] + dego_ref[1, :, 0:1]
    ns = jnp.where(deg_out > 0, lax.rsqrt(jnp.maximum(deg_out, 1.0)), 0.0)
    g = x * ns
    t1_ref[...] = g * keep0_ref[...]
    t2_ref[...] = g
    t3_ref[...] = x * keep1_ref[...]


def _prep_call(feat_pad, dego, keep0, keep1):
    sd = jax.ShapeDtypeStruct((N_PAD, D), jnp.float32)
    return pl.pallas_call(
        _prep_body, out_shape=(sd, sd, sd))(feat_pad, dego, keep0, keep1)


def _make_conv(use_nd):
    def body(p_ref, degi_ref, W_ref, b_ref, a_ref, h_ref, s_ref):
        p = p_ref[0] + p_ref[1]
        if use_nd:
            deg_in = degi_ref[0, :, 0:1] + degi_ref[1, :, 0:1]
            nd = jnp.where(deg_in > 0, lax.rsqrt(jnp.maximum(deg_in, 1.0)),
                           0.0)
            p = p * nd
        h = jnp.dot(p, W_ref[...], preferred_element_type=jnp.float32)
        h = h + b_ref[...]
        a = a_ref[0, 0]
        h = jnp.where(h >= 0, h, a * h)
        valid = (lax.broadcasted_iota(jnp.int32, (N_PAD, 1), 0)
                 < N).astype(jnp.float32)
        hv = h * valid
        s_ref[0:1, :] = jnp.sum(hv, axis=0, keepdims=True)
        s_ref[1:2, :] = jnp.sum(hv * h, axis=0, keepdims=True)
        h_ref[...] = h

    def call(p, degi, W, b, a):
        return pl.pallas_call(
            body,
            out_shape=(jax.ShapeDtypeStruct((N_PAD, D), jnp.float32),
                       jax.ShapeDtypeStruct((2, D), jnp.float32)),
        )(p, degi, W, b, a)

    return call


_conv_enc = _make_conv(True)
_conv_dec = _make_conv(False)


def _bn_act(h, s, gamma, beta, a):
    mu = s[0:1, :] * (1.0 / N)
    var = s[1:2, :] * (1.0 / N) - mu * mu
    z = (h - mu) * lax.rsqrt(var + BN_EPS) * gamma + beta
    return jnp.where(z >= 0, z, a * z)


def _cos_loss(z1, z2, maskf):
    n1 = jnp.maximum(jnp.sqrt(jnp.sum(z1 * z1, 1, keepdims=True)), 1e-12)
    n2 = jnp.maximum(jnp.sqrt(jnp.sum(z2 * z2, 1, keepdims=True)), 1e-12)
    d = jnp.sum(z1 * z2, 1, keepdims=True) / (n1 * n2)
    return jnp.sum(maskf * (1.0 - d)) * (1.0 / (N // 2))


BLKL = 2048


def _loss_body(h1_ref, s1_ref, h2_ref, s2_ref, h1b_ref, s1b_ref,
               h2b_ref, s2b_ref, g1_ref, be1_ref, aa1_ref, g2_ref, be2_ref,
               aa2_ref, m0_ref, m1_ref, out_ref):
    g1 = g1_ref[...]
    be1 = be1_ref[...]
    aa1 = aa1_ref[0, 0]
    g2 = g2_ref[...]
    be2 = be2_ref[...]
    aa2 = aa2_ref[0, 0]
    z1 = _bn_act(h1_ref[...], s1_ref[...], g1, be1, aa1)
    z2 = _bn_act(h2_ref[...], s2_ref[...], g2, be2, aa2)
    loss1 = _cos_loss(z1, z2, m0_ref[...])
    z1b = _bn_act(h1b_ref[...], s1b_ref[...], g1, be1, aa1)
    z2b = _bn_act(h2b_ref[...], s2b_ref[...], g2, be2, aa2)
    loss2 = _cos_loss(z1b, z2b, m1_ref[...])

    @pl.when(pl.program_id(0) == 0)
    def _init():
        out_ref[...] = jnp.zeros((1, 1), jnp.float32)

    out_ref[...] += (0.5 * loss1 + 0.5 * loss2).reshape(1, 1)


def kernel(feat, edge_index, diff_edge_index, edge_weight,
           W1, b1, a_conv1, gamma1, beta1, a_act1,
           W2, b2, a_conv2, gamma2, beta2, a_act2):
    f32 = jnp.float32
    mkey = jax.random.key(42)
    m0 = jax.random.permutation(jax.random.fold_in(mkey, 0), N)[: N // 2]
    m1 = jax.random.permutation(jax.random.fold_in(mkey, 1), N)[: N // 2]
    keep0 = jnp.ones((N_PAD, 1), f32).at[m0, 0].set(0.0)
    keep1 = jnp.ones((N_PAD, 1), f32).at[m1, 0].set(0.0)
    mask0 = jnp.zeros((N_PAD, 1), f32).at[m0, 0].set(1.0)
    mask1 = jnp.zeros((N_PAD, 1), f32).at[m1, 0].set(1.0)

    feat_pad = jnp.zeros((N_PAD, D), f32).at[:N].set(feat)
    epad = jnp.full((E_PAD - E,), N, jnp.int32)
    se = jnp.concatenate([edge_index[0], epad])
    de = jnp.concatenate([edge_index[1], epad])
    sd = jnp.concatenate([diff_edge_index[0], epad])
    dd = jnp.concatenate([diff_edge_index[1], epad])
    ewp = jnp.concatenate([edge_weight, jnp.zeros((E_PAD - E,), f32)])
    zeros_d = jnp.zeros((N_PAD, D), f32)
    ones_d = jnp.zeros((K, D), f32).at[:, 0].set(1.0)

    dego = _hist_kernel(se, ones_d, zeros_d).reshape(NC, N_PAD, D)
    degi = _hist_kernel(de, ones_d, zeros_d).reshape(NC, N_PAD, D)

    t_enc1, t_enc2, t_dec2 = _prep_call(feat_pad, dego, keep0, keep1)

    a_e1 = _agg_plain(t_enc1, se, de, ewp, zeros_d).reshape(NC, N_PAD, D)
    a_e2 = _agg_plain(t_enc2, se, de, ewp, zeros_d).reshape(NC, N_PAD, D)
    a_d1 = _agg_weighted(feat_pad, sd, dd, ewp, zeros_d).reshape(NC, N_PAD, D)
    a_d2 = _agg_weighted(t_dec2, sd, dd, ewp, zeros_d).reshape(NC, N_PAD, D)

    b1r = b1.reshape(1, D)
    b2r = b2.reshape(1, D)
    ac1 = a_conv1.reshape(1, 1)
    ac2 = a_conv2.reshape(1, 1)
    h1, s1 = _conv_enc(a_e1, degi, W1, b1r, ac1)
    h1b, s1b = _conv_enc(a_e2, degi, W1, b1r, ac1)
    h2, s2 = _conv_dec(a_d1, degi, W2, b2r, ac2)
    h2b, s2b = _conv_dec(a_d2, degi, W2, b2r, ac2)

    hspec = pl.BlockSpec((BLKL, D), lambda i: (i, 0))
    sspec = pl.BlockSpec((2, D), lambda i: (0, 0))
    vspec = pl.BlockSpec((1, D), lambda i: (0, 0))
    aspec = pl.BlockSpec((1, 1), lambda i: (0, 0))
    mspec = pl.BlockSpec((BLKL, 1), lambda i: (i, 0))
    out = pl.pallas_call(
        _loss_body,
        grid=(N_PAD // BLKL,),
        in_specs=[hspec, sspec, hspec, sspec, hspec, sspec, hspec, sspec,
                  vspec, vspec, aspec, vspec, vspec, aspec, mspec, mspec],
        out_specs=pl.BlockSpec((1, 1), lambda i: (0, 0)),
        out_shape=jax.ShapeDtypeStruct((1, 1), jnp.float32),
    )(h1, s1, h2, s2, h1b, s1b, h2b, s2b,
      gamma1.reshape(1, D), beta1.reshape(1, D), a_act1.reshape(1, 1),
      gamma2.reshape(1, D), beta2.reshape(1, D), a_act2.reshape(1, 1),
      mask0, mask1)
    return out[0, 0]


# speedup vs baseline: 2.7412x; 1.3784x over previous
import functools

import jax
import jax.numpy as jnp
from jax import lax
from jax.experimental import pallas as pl
from jax.experimental.pallas import tpu as pltpu
from jax.experimental.pallas import tpu_sc as plsc

N = 10000
D = 128
E = 320000
NC = 2
NS = 16
NW = NC * NS
N_PAD = 10240
E_PAD = 327680
EPW = E_PAD // NW
K = 128
RPT = N_PAD // NS
BN_EPS = 1e-5

_mesh = plsc.VectorSubcoreMesh(
    core_axis_name="c", subcore_axis_name="s", num_cores=NC, num_subcores=NS)


def _hist_body(idx_hbm, ones_hbm, zeros_hbm, out_hbm, idx_v, ones_v, acc, sem):
    cid = lax.axis_index("c")
    sid = lax.axis_index("s")
    r0 = sid * RPT
    cb = (cid * NS + sid) * (EPW // K)
    pltpu.sync_copy(zeros_hbm.at[pl.ds(r0, RPT)], acc.at[pl.ds(r0, RPT)])
    pltpu.sync_copy(ones_hbm, ones_v)
    pltpu.sync_copy(idx_hbm.at[pl.ds(cb, EPW // K)], idx_v)
    plsc.subcore_barrier()

    def chunk(i, carry):
        pltpu.sync_copy(ones_v, acc.at[idx_v.at[i]], add=True)
        return carry

    lax.fori_loop(0, EPW // K, chunk, 0)
    plsc.subcore_barrier()
    pltpu.sync_copy(acc.at[pl.ds(r0, RPT)],
                    out_hbm.at[pl.ds(cid * N_PAD + r0, RPT)])


_hist_kernel = functools.partial(
    pl.kernel,
    out_type=jax.ShapeDtypeStruct((NC * N_PAD, D), jnp.float32),
    mesh=_mesh,
    scratch_types=[
        pltpu.VMEM((EPW // K, K), jnp.int32),
        pltpu.VMEM((K, D), jnp.float32),
        pltpu.VMEM_SHARED((N_PAD, D), jnp.float32),
        pltpu.SemaphoreType.DMA,
    ])(_hist_body)


NCHUNK = EPW // K
NPAIR = NCHUNK // 2


def _make_agg(weighted):
    def body(table_hbm, src_hbm, dst_hbm, ew_hbm, zeros_hbm, out_hbm,
             sv0, sv1, dv0, dv1, wv0, wv1, rows0, rows1,
             acc, sem0, sem1, semi0, semi1):
        cid = lax.axis_index("c")
        sid = lax.axis_index("s")
        r0 = sid * RPT
        ebase = (cid * NS + sid) * EPW
        pltpu.sync_copy(zeros_hbm.at[pl.ds(r0, RPT)], acc.at[pl.ds(r0, RPT)])
        plsc.subcore_barrier()

        def fetch_idx(i, sv, dv, wv, semi):
            base = ebase + i * K
            pltpu.async_copy(src_hbm.at[pl.ds(base, K)], sv, semi)
            pltpu.async_copy(dst_hbm.at[pl.ds(base, K)], dv, semi)
            if weighted:
                pltpu.async_copy(ew_hbm.at[pl.ds(base, K)], wv, semi)

        def wait_idx(i, sv, dv, wv, semi):
            base = ebase + i * K
            pltpu.make_async_copy(src_hbm.at[pl.ds(base, K)], sv, semi).wait()
            pltpu.make_async_copy(dst_hbm.at[pl.ds(base, K)], dv, semi).wait()
            if weighted:
                pltpu.make_async_copy(ew_hbm.at[pl.ds(base, K)], wv,
                                      semi).wait()

        def wmul(wv, rows_v):
            def grp(g, c):
                w16 = wv[pl.ds(g * 16, 16)]
                for l in range(16):
                    j = g * 16 + l
                    wl = w16[l]
                    for t in range(D // 16):
                        rows_v[j, pl.ds(t * 16, 16)] = (
                            rows_v[j, pl.ds(t * 16, 16)] * wl)
                return c

            lax.fori_loop(0, K // 16, grp, 0)

        pltpu.sync_copy(src_hbm.at[pl.ds(ebase, K)], sv0)
        pltpu.sync_copy(dst_hbm.at[pl.ds(ebase, K)], dv0)
        if weighted:
            pltpu.sync_copy(ew_hbm.at[pl.ds(ebase, K)], wv0)
        pltpu.async_copy(table_hbm.at[sv0], rows0, sem0)
        fetch_idx(1, sv1, dv1, wv1, semi1)

        def pair(g, carry):
            i0 = 2 * g
            wait_idx(i0 + 1, sv1, dv1, wv1, semi1)
            pltpu.async_copy(table_hbm.at[sv1], rows1, sem1)
            pltpu.make_async_copy(table_hbm.at[sv0], rows0, sem0).wait()
            if weighted:
                wmul(wv0, rows0)
            pltpu.sync_copy(rows0, acc.at[dv0], add=True)

            @pl.when(i0 + 2 < NCHUNK)
            def _next0():
                fetch_idx(i0 + 2, sv0, dv0, wv0, semi0)
                wait_idx(i0 + 2, sv0, dv0, wv0, semi0)
                pltpu.async_copy(table_hbm.at[sv0], rows0, sem0)

            pltpu.make_async_copy(table_hbm.at[sv1], rows1, sem1).wait()
            if weighted:
                wmul(wv1, rows1)
            pltpu.sync_copy(rows1, acc.at[dv1], add=True)

            @pl.when(i0 + 3 < NCHUNK)
            def _next1():
                fetch_idx(i0 + 3, sv1, dv1, wv1, semi1)

            return carry

        lax.fori_loop(0, NPAIR, pair, 0)
        plsc.subcore_barrier()
        pltpu.sync_copy(acc.at[pl.ds(r0, RPT)],
                        out_hbm.at[pl.ds(cid * N_PAD + r0, RPT)])

    return functools.partial(
        pl.kernel,
        out_type=jax.ShapeDtypeStruct((NC * N_PAD, D), jnp.float32),
        mesh=_mesh,
        scratch_types=[
            pltpu.VMEM((K,), jnp.int32),
            pltpu.VMEM((K,), jnp.int32),
            pltpu.VMEM((K,), jnp.int32),
            pltpu.VMEM((K,), jnp.int32),
            pltpu.VMEM((K,), jnp.float32),
            pltpu.VMEM((K,), jnp.float32),
            pltpu.VMEM((K, D), jnp.float32),
            pltpu.VMEM((K, D), jnp.float32),
            pltpu.VMEM_SHARED((N_PAD, D), jnp.float32),
            pltpu.SemaphoreType.DMA,
            pltpu.SemaphoreType.DMA,
            pltpu.SemaphoreType.DMA,
            pltpu.SemaphoreType.DMA,
        ])(body)


_agg_plain = _make_agg(False)
_agg_weighted = _make_agg(True)


def _prep_body(feat_ref, dego_ref, keep0_ref, keep1_ref,
               t1_ref, t2_ref, t3_ref):
    x = feat_ref[...]
    deg_out = dego_ref[0, :, 0:1] + dego_ref[1, :, 0:1]
    ns = jnp.where(deg_out > 0, lax.rsqrt(jnp.maximum(deg_out, 1.0)), 0.0)
    g = x * ns
    t1_ref[...] = g * keep0_ref[...]
    t2_ref[...] = g
    t3_ref[...] = x * keep1_ref[...]


def _prep_call(feat_pad, dego, keep0, keep1):
    sd = jax.ShapeDtypeStruct((N_PAD, D), jnp.float32)
    return pl.pallas_call(
        _prep_body, out_shape=(sd, sd, sd))(feat_pad, dego, keep0, keep1)


def _make_conv(use_nd):
    def body(p_ref, degi_ref, W_ref, b_ref, a_ref, h_ref, s_ref):
        p = p_ref[0] + p_ref[1]
        if use_nd:
            deg_in = degi_ref[0, :, 0:1] + degi_ref[1, :, 0:1]
            nd = jnp.where(deg_in > 0, lax.rsqrt(jnp.maximum(deg_in, 1.0)),
                           0.0)
            p = p * nd
        h = jnp.dot(p, W_ref[...], preferred_element_type=jnp.float32)
        h = h + b_ref[...]
        a = a_ref[0, 0]
        h = jnp.where(h >= 0, h, a * h)
        valid = (lax.broadcasted_iota(jnp.int32, (N_PAD, 1), 0)
                 < N).astype(jnp.float32)
        hv = h * valid
        s_ref[0:1, :] = jnp.sum(hv, axis=0, keepdims=True)
        s_ref[1:2, :] = jnp.sum(hv * h, axis=0, keepdims=True)
        h_ref[...] = h

    def call(p, degi, W, b, a):
        return pl.pallas_call(
            body,
            out_shape=(jax.ShapeDtypeStruct((N_PAD, D), jnp.float32),
                       jax.ShapeDtypeStruct((2, D), jnp.float32)),
        )(p, degi, W, b, a)

    return call


_conv_enc = _make_conv(True)
_conv_dec = _make_conv(False)


def _bn_act(h, s, gamma, beta, a):
    mu = s[0:1, :] * (1.0 / N)
    var = s[1:2, :] * (1.0 / N) - mu * mu
    z = (h - mu) * lax.rsqrt(var + BN_EPS) * gamma + beta
    return jnp.where(z >= 0, z, a * z)


def _cos_loss(z1, z2, maskf):
    n1 = jnp.maximum(jnp.sqrt(jnp.sum(z1 * z1, 1, keepdims=True)), 1e-12)
    n2 = jnp.maximum(jnp.sqrt(jnp.sum(z2 * z2, 1, keepdims=True)), 1e-12)
    d = jnp.sum(z1 * z2, 1, keepdims=True) / (n1 * n2)
    return jnp.sum(maskf * (1.0 - d)) * (1.0 / (N // 2))


BLKL = 2048


def _loss_body(h1_ref, s1_ref, h2_ref, s2_ref, h1b_ref, s1b_ref,
               h2b_ref, s2b_ref, g1_ref, be1_ref, aa1_ref, g2_ref, be2_ref,
               aa2_ref, m0_ref, m1_ref, out_ref):
    g1 = g1_ref[...]
    be1 = be1_ref[...]
    aa1 = aa1_ref[0, 0]
    g2 = g2_ref[...]
    be2 = be2_ref[...]
    aa2 = aa2_ref[0, 0]
    z1 = _bn_act(h1_ref[...], s1_ref[...], g1, be1, aa1)
    z2 = _bn_act(h2_ref[...], s2_ref[...], g2, be2, aa2)
    loss1 = _cos_loss(z1, z2, m0_ref[...])
    z1b = _bn_act(h1b_ref[...], s1b_ref[...], g1, be1, aa1)
    z2b = _bn_act(h2b_ref[...], s2b_ref[...], g2, be2, aa2)
    loss2 = _cos_loss(z1b, z2b, m1_ref[...])

    @pl.when(pl.program_id(0) == 0)
    def _init():
        out_ref[...] = jnp.zeros((1, 1), jnp.float32)

    out_ref[...] += (0.5 * loss1 + 0.5 * loss2).reshape(1, 1)


def kernel(feat, edge_index, diff_edge_index, edge_weight,
           W1, b1, a_conv1, gamma1, beta1, a_act1,
           W2, b2, a_conv2, gamma2, beta2, a_act2):
    f32 = jnp.float32
    mkey = jax.random.key(42)
    m0 = jax.random.permutation(jax.random.fold_in(mkey, 0), N)[: N // 2]
    m1 = jax.random.permutation(jax.random.fold_in(mkey, 1), N)[: N // 2]
    keep0 = jnp.ones((N_PAD, 1), f32).at[m0, 0].set(0.0)
    keep1 = jnp.ones((N_PAD, 1), f32).at[m1, 0].set(0.0)
    mask0 = jnp.zeros((N_PAD, 1), f32).at[m0, 0].set(1.0)
    mask1 = jnp.zeros((N_PAD, 1), f32).at[m1, 0].set(1.0)

    feat_pad = jnp.zeros((N_PAD, D), f32).at[:N].set(feat)
    epad = jnp.full((E_PAD - E,), N, jnp.int32)
    nch = E_PAD // K
    se = jnp.concatenate([edge_index[0], epad])
    de = jnp.concatenate([edge_index[1], epad])
    sd = jnp.concatenate([diff_edge_index[0], epad])
    dd = jnp.concatenate([diff_edge_index[1], epad])
    ewp = jnp.concatenate([edge_weight, jnp.zeros((E_PAD - E,), f32)])
    zeros_d = jnp.zeros((N_PAD, D), f32)
    ones_d = jnp.zeros((K, D), f32).at[:, 0].set(1.0)

    dego = _hist_kernel(se.reshape(nch, K), ones_d,
                        zeros_d).reshape(NC, N_PAD, D)
    degi = _hist_kernel(de.reshape(nch, K), ones_d,
                        zeros_d).reshape(NC, N_PAD, D)

    t_enc1, t_enc2, t_dec2 = _prep_call(feat_pad, dego, keep0, keep1)

    a_e1 = _agg_plain(t_enc1, se, de, ewp, zeros_d).reshape(NC, N_PAD, D)
    a_e2 = _agg_plain(t_enc2, se, de, ewp, zeros_d).reshape(NC, N_PAD, D)
    a_d1 = _agg_weighted(feat_pad, sd, dd, ewp, zeros_d).reshape(NC, N_PAD, D)
    a_d2 = _agg_weighted(t_dec2, sd, dd, ewp, zeros_d).reshape(NC, N_PAD, D)

    b1r = b1.reshape(1, D)
    b2r = b2.reshape(1, D)
    ac1 = a_conv1.reshape(1, 1)
    ac2 = a_conv2.reshape(1, 1)
    h1, s1 = _conv_enc(a_e1, degi, W1, b1r, ac1)
    h1b, s1b = _conv_enc(a_e2, degi, W1, b1r, ac1)
    h2, s2 = _conv_dec(a_d1, degi, W2, b2r, ac2)
    h2b, s2b = _conv_dec(a_d2, degi, W2, b2r, ac2)

    hspec = pl.BlockSpec((BLKL, D), lambda i: (i, 0))
    sspec = pl.BlockSpec((2, D), lambda i: (0, 0))
    vspec = pl.BlockSpec((1, D), lambda i: (0, 0))
    aspec = pl.BlockSpec((1, 1), lambda i: (0, 0))
    mspec = pl.BlockSpec((BLKL, 1), lambda i: (i, 0))
    out = pl.pallas_call(
        _loss_body,
        grid=(N_PAD // BLKL,),
        in_specs=[hspec, sspec, hspec, sspec, hspec, sspec, hspec, sspec,
                  vspec, vspec, aspec, vspec, vspec, aspec, mspec, mspec],
        out_specs=pl.BlockSpec((1, 1), lambda i: (0, 0)),
        out_shape=jax.ShapeDtypeStruct((1, 1), jnp.float32),
    )(h1, s1, h2, s2, h1b, s1b, h2b, s2b,
      gamma1.reshape(1, D), beta1.reshape(1, D), a_act1.reshape(1, 1),
      gamma2.reshape(1, D), beta2.reshape(1, D), a_act2.reshape(1, 1),
      mask0, mask1)
    return out[0, 0]
